# baseline (speedup 1.0000x reference)
import functools

import jax
import jax.numpy as jnp
from jax.experimental import pallas as pl
from jax.experimental.pallas import tpu as pltpu

L = 17
B = 128
S = 512


def _viterbi_step(dp, atb_ref):
    vals = [dp[k:k + 1, :] + atb_ref[k] for k in range(L)]
    idxs = list(range(L))
    while len(vals) > 1:
        nv, ni = [], []
        for i in range(0, len(vals) - 1, 2):
            a, b = vals[i], vals[i + 1]
            ia, ib = idxs[i], idxs[i + 1]
            if isinstance(ia, int):
                ia = jnp.full((L, B), ia, jnp.int32)
            if isinstance(ib, int):
                ib = jnp.full((L, B), ib, jnp.int32)
            gt = b > a
            nv.append(jnp.maximum(a, b))
            ni.append(jnp.where(gt, ib, ia))
        if len(vals) % 2:
            nv.append(vals[-1])
            ni.append(idxs[-1])
        vals, idxs = nv, ni
    return vals[0], idxs[0]


def _crf_kernel(xt_ref, yt_ref, A_ref, AT_ref, path_ref, nll_ref,
                bp_ref, atb_ref):
    A = A_ref[...]
    AT = AT_ref[...]
    E = jnp.exp(AT)

    for k in range(L):
        atb_ref[k] = jnp.broadcast_to(AT[:, k:k + 1], (L, B))

    lane_iota = jax.lax.broadcasted_iota(jnp.int32, (L, B), 0)

    x0 = xt_ref[0]
    y0 = yt_ref[pl.ds(0, 1), :]

    dp0 = x0
    m0 = jnp.max(x0, axis=0, keepdims=True)
    w0 = jax.lax.dot(E, jnp.exp(x0 - m0),
                     preferred_element_type=jnp.float32)
    acc0 = jnp.where(lane_iota == y0, x0, 0.0)

    def gold(acc, yprev, yj, xj):
        ohprev = (lane_iota == yprev).astype(jnp.float32)
        acols = jax.lax.dot(A, ohprev, preferred_element_type=jnp.float32)
        return acc + jnp.where(lane_iota == yj, xj + acols, 0.0)

    def step2(t, carry):
        dp, w, logacc, acc, yprev = carry
        j1 = 2 * t + 1
        j2 = 2 * t + 2
        x1 = xt_ref[j1]
        x2 = xt_ref[j2]
        y1 = yt_ref[pl.ds(j1, 1), :]
        y2 = yt_ref[pl.ds(j2, 1), :]

        best1, bp1 = _viterbi_step(dp, atb_ref)
        bp_ref[j1] = bp1
        dp1 = best1 + x1
        best2, bp2 = _viterbi_step(dp1, atb_ref)
        bp_ref[j2] = bp2
        dp2 = best2 + x2

        s = jnp.max(w, axis=0, keepdims=True)
        rs = 1.0 / s
        wh = jax.lax.dot(E, w * jnp.exp(x1),
                         preferred_element_type=jnp.float32)
        wn = jax.lax.dot(E, wh * jnp.exp(x2),
                         preferred_element_type=jnp.float32) * rs
        logacc_n = logacc + jnp.log(s)

        acc = gold(acc, yprev, y1, x1)
        acc = gold(acc, y1, y2, x2)
        return dp2, wn, logacc_n, acc, y2

    dp, w, logacc, acc, yprev = jax.lax.fori_loop(
        0, (S - 2) // 2, step2, (dp0, w0, m0, acc0, y0))

    xl = xt_ref[S - 1]
    yl = yt_ref[pl.ds(S - 1, 1), :]
    best, besti = _viterbi_step(dp, atb_ref)
    bp_ref[S - 1] = besti
    dp_last = best + xl

    alpha = logacc + jnp.log(w)
    v = xl + alpha
    mz = jnp.max(v, axis=0, keepdims=True)
    z = mz + jnp.log(jnp.sum(jnp.exp(v - mz), axis=0, keepdims=True))

    acc = gold(acc, yprev, yl, xl)

    s = jnp.sum(acc, axis=0, keepdims=True)
    nll_ref[...] = jnp.sum(z - s, axis=1, keepdims=True) * (1.0 / B)

    last = jnp.zeros((1, B), jnp.int32)
    bestv = dp_last[0:1, :]
    for k in range(1, L):
        row = dp_last[k:k + 1, :]
        gt = row > bestv
        bestv = jnp.where(gt, row, bestv)
        last = jnp.where(gt, k, last)
    path_ref[pl.ds(S - 1, 1), :] = last

    def back(t, cur):
        j = S - 1 - t
        bprow = bp_ref[j]
        prev = jnp.max(jnp.where(lane_iota == cur, bprow, 0),
                       axis=0, keepdims=True)
        path_ref[pl.ds(j - 1, 1), :] = prev
        return prev

    jax.lax.fori_loop(0, S - 1, back, last)


@functools.partial(jax.jit, static_argnames=())
def kernel(x, y, A):
    xt = jnp.transpose(x, (1, 2, 0))
    yt = jnp.transpose(y, (1, 0))
    AT = jnp.transpose(A, (1, 0))

    path_t, nll = pl.pallas_call(
        _crf_kernel,
        out_shape=(
            jax.ShapeDtypeStruct((S, B), jnp.int32),
            jax.ShapeDtypeStruct((1, 1), jnp.float32),
        ),
        scratch_shapes=[
            pltpu.VMEM((S, L, B), jnp.int32),
            pltpu.VMEM((L, L, B), jnp.float32),
        ],
    )(xt, yt, A, AT)

    return path_t.T, nll[0, 0]


# speedup vs baseline: 1.7392x; 1.7392x over previous
import functools

import jax
import jax.numpy as jnp
from jax.experimental import pallas as pl
from jax.experimental.pallas import tpu as pltpu

L = 17
B = 128
S = 512


def _tree(items, combine):
    while len(items) > 1:
        nxt = [combine(items[i], items[i + 1])
               for i in range(0, len(items) - 1, 2)]
        if len(items) % 2:
            nxt.append(items[-1])
        items = nxt
    return items[0]


def _viterbi_step(dp, atb_ref, idxt_ref):
    vals = [dp[k:k + 1, :] + atb_ref[k] for k in range(L)]
    idxs = list(range(L))
    while len(vals) > 1:
        nv, ni = [], []
        for i in range(0, len(vals) - 1, 2):
            a, b = vals[i], vals[i + 1]
            ia, ib = idxs[i], idxs[i + 1]
            if isinstance(ia, int):
                ia = idxt_ref[ia]
            if isinstance(ib, int):
                ib = idxt_ref[ib]
            gt = b > a
            nv.append(jnp.maximum(a, b))
            ni.append(jnp.where(gt, ib, ia))
        if len(vals) % 2:
            nv.append(vals[-1])
            ni.append(idxs[-1])
        vals, idxs = nv, ni
    return vals[0], idxs[0]


def _fwd_step(u, ebt_ref):
    terms = [u[l:l + 1, :] * ebt_ref[l] for l in range(L)]
    return _tree(terms, jnp.add)


def _crf_kernel(xt_ref, yt_ref, A_ref, AT_ref, path_ref, nll_ref,
                bp_ref, atb_ref, ebt_ref, idxt_ref):
    A = A_ref[...]
    AT = AT_ref[...]
    E = jnp.exp(AT)

    for k in range(L):
        atb_ref[k] = jnp.broadcast_to(AT[:, k:k + 1], (L, B))
        ebt_ref[k] = jnp.broadcast_to(E[:, k:k + 1], (L, B))
        idxt_ref[k] = jnp.full((L, B), k, jnp.int32)

    lane_iota = jax.lax.broadcasted_iota(jnp.int32, (L, B), 0)

    x0 = xt_ref[0]
    y0 = yt_ref[pl.ds(0, 1), :]

    dp0 = x0
    m0 = jnp.max(x0, axis=0, keepdims=True)
    w0 = _fwd_step(jnp.exp(x0 - m0), ebt_ref)
    acc0 = jnp.where(lane_iota == y0, x0, 0.0)

    def gold(acc, yprev, yj, xj):
        ohprev = (lane_iota == yprev).astype(jnp.float32)
        acols = jax.lax.dot(A, ohprev, preferred_element_type=jnp.float32)
        return acc + jnp.where(lane_iota == yj, xj + acols, 0.0)

    def step2(t, carry):
        dp, w, logacc, acc, yprev = carry
        j1 = 2 * t + 1
        j2 = 2 * t + 2
        x1 = xt_ref[j1]
        x2 = xt_ref[j2]
        y1 = yt_ref[pl.ds(j1, 1), :]
        y2 = yt_ref[pl.ds(j2, 1), :]

        best1, bp1 = _viterbi_step(dp, atb_ref, idxt_ref)
        bp_ref[j1] = bp1
        dp1 = best1 + x1
        best2, bp2 = _viterbi_step(dp1, atb_ref, idxt_ref)
        bp_ref[j2] = bp2
        dp2 = best2 + x2

        s = jnp.max(w, axis=0, keepdims=True)
        rs = 1.0 / s
        wh = _fwd_step(w * jnp.exp(x1), ebt_ref)
        wn = _fwd_step(wh * jnp.exp(x2), ebt_ref) * rs
        logacc_n = logacc + jnp.log(s)

        acc = gold(acc, yprev, y1, x1)
        acc = gold(acc, y1, y2, x2)
        return dp2, wn, logacc_n, acc, y2

    dp, w, logacc, acc, yprev = jax.lax.fori_loop(
        0, (S - 2) // 2, step2, (dp0, w0, m0, acc0, y0))

    xl = xt_ref[S - 1]
    yl = yt_ref[pl.ds(S - 1, 1), :]
    best, besti = _viterbi_step(dp, atb_ref, idxt_ref)
    bp_ref[S - 1] = besti
    dp_last = best + xl

    alpha = logacc + jnp.log(w)
    v = xl + alpha
    mz = jnp.max(v, axis=0, keepdims=True)
    z = mz + jnp.log(jnp.sum(jnp.exp(v - mz), axis=0, keepdims=True))

    acc = gold(acc, yprev, yl, xl)

    s = jnp.sum(acc, axis=0, keepdims=True)
    nll_ref[...] = jnp.sum(z - s, axis=1, keepdims=True) * (1.0 / B)

    last = jnp.zeros((1, B), jnp.int32)
    bestv = dp_last[0:1, :]
    for k in range(1, L):
        row = dp_last[k:k + 1, :]
        gt = row > bestv
        bestv = jnp.where(gt, row, bestv)
        last = jnp.where(gt, k, last)
    path_ref[pl.ds(S - 1, 1), :] = last

    def back(t, cur):
        j = S - 1 - t
        bprow = bp_ref[j]
        prev = jnp.max(jnp.where(lane_iota == cur, bprow, 0),
                       axis=0, keepdims=True)
        path_ref[pl.ds(j - 1, 1), :] = prev
        return prev

    jax.lax.fori_loop(0, S - 1, back, last)


@functools.partial(jax.jit, static_argnames=())
def kernel(x, y, A):
    xt = jnp.transpose(x, (1, 2, 0))
    yt = jnp.transpose(y, (1, 0))
    AT = jnp.transpose(A, (1, 0))

    path_t, nll = pl.pallas_call(
        _crf_kernel,
        out_shape=(
            jax.ShapeDtypeStruct((S, B), jnp.int32),
            jax.ShapeDtypeStruct((1, 1), jnp.float32),
        ),
        scratch_shapes=[
            pltpu.VMEM((S, L, B), jnp.int32),
            pltpu.VMEM((L, L, B), jnp.float32),
            pltpu.VMEM((L, L, B), jnp.float32),
            pltpu.VMEM((L, L, B), jnp.int32),
        ],
    )(xt, yt, A, AT)

    return path_t.T, nll[0, 0]


# speedup vs baseline: 1.7854x; 1.0266x over previous
import functools

import jax
import jax.numpy as jnp
from jax.experimental import pallas as pl
from jax.experimental.pallas import tpu as pltpu

L = 17
B = 128
S = 512


def _tree(items, combine):
    while len(items) > 1:
        nxt = [combine(items[i], items[i + 1])
               for i in range(0, len(items) - 1, 2)]
        if len(items) % 2:
            nxt.append(items[-1])
        items = nxt
    return items[0]


def _viterbi_step(dp, atb_ref, idxt_ref):
    vals = [dp[k:k + 1, :] + atb_ref[k] for k in range(L)]
    idxs = list(range(L))
    while len(vals) > 1:
        nv, ni = [], []
        for i in range(0, len(vals) - 1, 2):
            a, b = vals[i], vals[i + 1]
            ia, ib = idxs[i], idxs[i + 1]
            if isinstance(ia, int):
                ia = idxt_ref[ia]
            if isinstance(ib, int):
                ib = idxt_ref[ib]
            gt = b > a
            nv.append(jnp.maximum(a, b))
            ni.append(jnp.where(gt, ib, ia))
        if len(vals) % 2:
            nv.append(vals[-1])
            ni.append(idxs[-1])
        vals, idxs = nv, ni
    return vals[0], idxs[0]


def _fwd_step(u, ebt_ref):
    terms = [u[l:l + 1, :] * ebt_ref[l] for l in range(L)]
    return _tree(terms, jnp.add)


def _crf_kernel(xt_ref, yt_ref, A_ref, AT_ref, path_ref, nll_ref,
                bp_ref, atb_ref, ebt_ref, idxt_ref):
    A = A_ref[...]
    AT = AT_ref[...]
    E = jnp.exp(AT)

    for k in range(L):
        atb_ref[k] = jnp.broadcast_to(AT[:, k:k + 1], (L, B))
        ebt_ref[k] = jnp.broadcast_to(E[:, k:k + 1], (L, B))
        idxt_ref[k] = jnp.full((L, B), k, jnp.int32)

    lane_iota = jax.lax.broadcasted_iota(jnp.int32, (L, B), 0)

    x0 = xt_ref[0]
    y0 = yt_ref[pl.ds(0, 1), :]

    dp0 = x0
    m0 = jnp.max(x0, axis=0, keepdims=True)
    w0 = _fwd_step(jnp.exp(x0 - m0), ebt_ref)
    acc0 = jnp.where(lane_iota == y0, x0, 0.0)

    def gold(acc, yprev, yj, xj):
        ohprev = (lane_iota == yprev).astype(jnp.float32)
        acols = jax.lax.dot(A, ohprev, preferred_element_type=jnp.float32)
        return acc + jnp.where(lane_iota == yj, xj + acols, 0.0)

    def step4(t, carry):
        dp, w, logacc, acc, yprev = carry
        j0 = 4 * t + 1
        xs = [xt_ref[j0 + d] for d in range(4)]
        ys = [yt_ref[pl.ds(j0 + d, 1), :] for d in range(4)]

        for d in range(4):
            best, bp = _viterbi_step(dp, atb_ref, idxt_ref)
            bp_ref[j0 + d] = bp
            dp = best + xs[d]

        s = jnp.max(w, axis=0, keepdims=True)
        rs = 1.0 / s
        for d in range(4):
            w = _fwd_step(w * jnp.exp(xs[d]), ebt_ref)
        w = w * rs
        logacc = logacc + jnp.log(s)

        for d in range(4):
            acc = gold(acc, yprev, ys[d], xs[d])
            yprev = ys[d]
        return dp, w, logacc, acc, yprev

    dp, w, logacc, acc, yprev = jax.lax.fori_loop(
        0, (S - 2) // 4, step4, (dp0, w0, m0, acc0, y0))

    for j in range(4 * ((S - 2) // 4) + 1, S - 1):
        xj = xt_ref[j]
        yj = yt_ref[pl.ds(j, 1), :]
        best, bp = _viterbi_step(dp, atb_ref, idxt_ref)
        bp_ref[j] = bp
        dp = best + xj
        s = jnp.max(w, axis=0, keepdims=True)
        w = _fwd_step(w * jnp.exp(xj), ebt_ref) * (1.0 / s)
        logacc = logacc + jnp.log(s)
        acc = gold(acc, yprev, yj, xj)
        yprev = yj

    xl = xt_ref[S - 1]
    yl = yt_ref[pl.ds(S - 1, 1), :]
    best, besti = _viterbi_step(dp, atb_ref, idxt_ref)
    bp_ref[S - 1] = besti
    dp_last = best + xl

    alpha = logacc + jnp.log(w)
    v = xl + alpha
    mz = jnp.max(v, axis=0, keepdims=True)
    z = mz + jnp.log(jnp.sum(jnp.exp(v - mz), axis=0, keepdims=True))

    acc = gold(acc, yprev, yl, xl)

    s = jnp.sum(acc, axis=0, keepdims=True)
    nll_ref[...] = jnp.sum(z - s, axis=1, keepdims=True) * (1.0 / B)

    last = jnp.zeros((1, B), jnp.int32)
    bestv = dp_last[0:1, :]
    for k in range(1, L):
        row = dp_last[k:k + 1, :]
        gt = row > bestv
        bestv = jnp.where(gt, row, bestv)
        last = jnp.where(gt, k, last)
    path_ref[pl.ds(S - 1, 1), :] = last

    def back1(t, cur):
        j = S - 1 - t
        bprow = bp_ref[j]
        prev = jnp.max(jnp.where(lane_iota == cur, bprow, 0),
                       axis=0, keepdims=True)
        path_ref[pl.ds(j - 1, 1), :] = prev
        return prev

    def back3(u, cur):
        for d in range(3):
            cur = back1(3 * u + d, cur)
        return cur

    cur = jax.lax.fori_loop(0, (S - 1) // 3, back3, last)
    for t in range(3 * ((S - 1) // 3), S - 1):
        cur = back1(t, cur)


@functools.partial(jax.jit, static_argnames=())
def kernel(x, y, A):
    xt = jnp.transpose(x, (1, 2, 0))
    yt = jnp.transpose(y, (1, 0))
    AT = jnp.transpose(A, (1, 0))

    path_t, nll = pl.pallas_call(
        _crf_kernel,
        out_shape=(
            jax.ShapeDtypeStruct((S, B), jnp.int32),
            jax.ShapeDtypeStruct((1, 1), jnp.float32),
        ),
        scratch_shapes=[
            pltpu.VMEM((S, L, B), jnp.int32),
            pltpu.VMEM((L, L, B), jnp.float32),
            pltpu.VMEM((L, L, B), jnp.float32),
            pltpu.VMEM((L, L, B), jnp.int32),
        ],
    )(xt, yt, A, AT)

    return path_t.T, nll[0, 0]


# speedup vs baseline: 1.8684x; 1.0465x over previous
import functools

import jax
import jax.numpy as jnp
from jax.experimental import pallas as pl
from jax.experimental.pallas import tpu as pltpu

L = 17
B = 128
S = 512


def _tree(items, combine):
    while len(items) > 1:
        nxt = [combine(items[i], items[i + 1])
               for i in range(0, len(items) - 1, 2)]
        if len(items) % 2:
            nxt.append(items[-1])
        items = nxt
    return items[0]


def _viterbi_step(dp, atb_ref, idxt_ref):
    vals = [dp[k:k + 1, :] + atb_ref[k] for k in range(L)]
    idxs = list(range(L))
    while len(vals) > 1:
        nv, ni = [], []
        for i in range(0, len(vals) - 1, 2):
            a, b = vals[i], vals[i + 1]
            ia, ib = idxs[i], idxs[i + 1]
            if isinstance(ia, int):
                ia = idxt_ref[ia]
            if isinstance(ib, int):
                ib = idxt_ref[ib]
            gt = b > a
            nv.append(jnp.maximum(a, b))
            ni.append(jnp.where(gt, ib, ia))
        if len(vals) % 2:
            nv.append(vals[-1])
            ni.append(idxs[-1])
        vals, idxs = nv, ni
    return vals[0], idxs[0]


def _fwd_step(u, ebt_ref):
    terms = [u[l:l + 1, :] * ebt_ref[l] for l in range(L)]
    return _tree(terms, jnp.add)


def _crf_kernel(xt_ref, yt_ref, A_ref, AT_ref, path_ref, nll_ref,
                bp_ref, atb_ref, ebt_ref, idxt_ref):
    A = A_ref[...]
    AT = AT_ref[...]
    E = jnp.exp(AT)

    for k in range(L):
        atb_ref[k] = jnp.broadcast_to(AT[:, k:k + 1], (L, B))
        ebt_ref[k] = jnp.broadcast_to(E[:, k:k + 1], (L, B))
        idxt_ref[k] = jnp.full((L, B), k, jnp.int32)

    lane_iota = jax.lax.broadcasted_iota(jnp.int32, (L, B), 0)

    x0 = xt_ref[0]
    y0 = yt_ref[pl.ds(0, 1), :]

    dp0 = x0
    m0 = jnp.max(x0, axis=0, keepdims=True)
    w0 = _fwd_step(jnp.exp(x0 - m0), ebt_ref)
    acc0 = jnp.where(lane_iota == y0, x0, 0.0)

    def gold(acc, yprev, yj, xj):
        ohprev = (lane_iota == yprev).astype(jnp.float32)
        acols = jax.lax.dot(A, ohprev, preferred_element_type=jnp.float32)
        return acc + jnp.where(lane_iota == yj, xj + acols, 0.0)

    def quad(q, carry):
        dp, w, logacc, acc, yprev = carry
        j0 = 4 * q + 1
        xs = [xt_ref[j0 + d] for d in range(4)]
        ys = [yt_ref[pl.ds(j0 + d, 1), :] for d in range(4)]

        for d in range(4):
            best, bp = _viterbi_step(dp, atb_ref, idxt_ref)
            bp_ref[j0 + d] = bp
            dp = best + xs[d]

        s = jnp.max(w, axis=0, keepdims=True)
        rs = 1.0 / s
        for d in range(4):
            w = _fwd_step(w * jnp.exp(xs[d]), ebt_ref)
        w = w * rs
        logacc = logacc + jnp.log(s)

        for d in range(4):
            acc = gold(acc, yprev, ys[d], xs[d])
            yprev = ys[d]
        return dp, w, logacc, acc, yprev

    def step8(t, carry):
        return quad(2 * t + 1, quad(2 * t, carry))

    carry = jax.lax.fori_loop(
        0, (S - 2) // 8, step8, (dp0, w0, m0, acc0, y0))
    dp, w, logacc, acc, yprev = quad((S - 2) // 4 - 1, carry)

    for j in range(4 * ((S - 2) // 4) + 1, S - 1):
        xj = xt_ref[j]
        yj = yt_ref[pl.ds(j, 1), :]
        best, bp = _viterbi_step(dp, atb_ref, idxt_ref)
        bp_ref[j] = bp
        dp = best + xj
        s = jnp.max(w, axis=0, keepdims=True)
        w = _fwd_step(w * jnp.exp(xj), ebt_ref) * (1.0 / s)
        logacc = logacc + jnp.log(s)
        acc = gold(acc, yprev, yj, xj)
        yprev = yj

    xl = xt_ref[S - 1]
    yl = yt_ref[pl.ds(S - 1, 1), :]
    best, besti = _viterbi_step(dp, atb_ref, idxt_ref)
    bp_ref[S - 1] = besti
    dp_last = best + xl

    alpha = logacc + jnp.log(w)
    v = xl + alpha
    mz = jnp.max(v, axis=0, keepdims=True)
    z = mz + jnp.log(jnp.sum(jnp.exp(v - mz), axis=0, keepdims=True))

    acc = gold(acc, yprev, yl, xl)

    s = jnp.sum(acc, axis=0, keepdims=True)
    nll_ref[...] = jnp.sum(z - s, axis=1, keepdims=True) * (1.0 / B)

    last = jnp.zeros((1, B), jnp.int32)
    bestv = dp_last[0:1, :]
    for k in range(1, L):
        row = dp_last[k:k + 1, :]
        gt = row > bestv
        bestv = jnp.where(gt, row, bestv)
        last = jnp.where(gt, k, last)
    path_ref[pl.ds(S - 1, 1), :] = last

    def back1(t, cur):
        j = S - 1 - t
        bprow = bp_ref[j]
        prev = jnp.max(jnp.where(lane_iota == cur, bprow, 0),
                       axis=0, keepdims=True)
        path_ref[pl.ds(j - 1, 1), :] = prev
        return prev

    def back3(u, cur):
        for d in range(3):
            cur = back1(3 * u + d, cur)
        return cur

    cur = jax.lax.fori_loop(0, (S - 1) // 3, back3, last)
    for t in range(3 * ((S - 1) // 3), S - 1):
        cur = back1(t, cur)


@functools.partial(jax.jit, static_argnames=())
def kernel(x, y, A):
    xt = jnp.transpose(x, (1, 2, 0))
    yt = jnp.transpose(y, (1, 0))
    AT = jnp.transpose(A, (1, 0))

    path_t, nll = pl.pallas_call(
        _crf_kernel,
        out_shape=(
            jax.ShapeDtypeStruct((S, B), jnp.int32),
            jax.ShapeDtypeStruct((1, 1), jnp.float32),
        ),
        scratch_shapes=[
            pltpu.VMEM((S, L, B), jnp.int32),
            pltpu.VMEM((L, L, B), jnp.float32),
            pltpu.VMEM((L, L, B), jnp.float32),
            pltpu.VMEM((L, L, B), jnp.int32),
        ],
    )(xt, yt, A, AT)

    return path_t.T, nll[0, 0]


# speedup vs baseline: 1.9196x; 1.0274x over previous
import functools

import jax
import jax.numpy as jnp
from jax.experimental import pallas as pl
from jax.experimental.pallas import tpu as pltpu

L = 17
B = 128
S = 512


def _tree(items, combine):
    while len(items) > 1:
        nxt = [combine(items[i], items[i + 1])
               for i in range(0, len(items) - 1, 2)]
        if len(items) % 2:
            nxt.append(items[-1])
        items = nxt
    return items[0]


def _viterbi_step(dp, atb_ref, idxt_ref):
    vals = [dp[k:k + 1, :] + atb_ref[k] for k in range(L)]
    idxs = list(range(L))
    while len(vals) > 1:
        nv, ni = [], []
        for i in range(0, len(vals) - 1, 2):
            a, b = vals[i], vals[i + 1]
            ia, ib = idxs[i], idxs[i + 1]
            if isinstance(ia, int):
                ia = idxt_ref[ia]
            if isinstance(ib, int):
                ib = idxt_ref[ib]
            gt = b > a
            nv.append(jnp.maximum(a, b))
            ni.append(jnp.where(gt, ib, ia))
        if len(vals) % 2:
            nv.append(vals[-1])
            ni.append(idxs[-1])
        vals, idxs = nv, ni
    return vals[0], idxs[0]


def _fwd_step(u, ebt_ref):
    terms = [u[l:l + 1, :] * ebt_ref[l] for l in range(L)]
    return _tree(terms, jnp.add)


def _crf_kernel(xt_ref, yt_ref, A_ref, AT_ref, path_ref, nll_ref,
                bp_ref, atb_ref, ebt_ref, idxt_ref):
    A = A_ref[...]
    AT = AT_ref[...]
    E = jnp.exp(AT)

    for k in range(L):
        atb_ref[k] = jnp.broadcast_to(AT[:, k:k + 1], (L, B))
        ebt_ref[k] = jnp.broadcast_to(E[:, k:k + 1], (L, B))
        idxt_ref[k] = jnp.full((L, B), k, jnp.int32)

    lane_iota = jax.lax.broadcasted_iota(jnp.int32, (L, B), 0)

    x0 = xt_ref[0]
    y0 = yt_ref[pl.ds(0, 1), :]

    dp0 = x0
    m0 = jnp.max(x0, axis=0, keepdims=True)
    w0 = _fwd_step(jnp.exp(x0 - m0), ebt_ref)
    acc0 = jnp.where(lane_iota == y0, x0, 0.0)

    def gold(acc, yprev, yj, xj):
        ohprev = (lane_iota == yprev).astype(jnp.float32)
        acols = jax.lax.dot(A, ohprev, preferred_element_type=jnp.float32)
        return acc + jnp.where(lane_iota == yj, xj + acols, 0.0)

    def quad(q, carry):
        dp, w, logacc, acc, yprev = carry
        j0 = 4 * q + 1
        xs = [xt_ref[j0 + d] for d in range(4)]
        ys = [yt_ref[pl.ds(j0 + d, 1), :] for d in range(4)]

        for d in range(4):
            best, bp = _viterbi_step(dp, atb_ref, idxt_ref)
            bp_ref[j0 + d] = bp
            dp = best + xs[d]

        s = jnp.max(w, axis=0, keepdims=True)
        rs = 1.0 / s
        for d in range(4):
            w = _fwd_step(w * jnp.exp(xs[d]), ebt_ref)
        w = w * rs
        logacc = logacc + jnp.log(s)

        for d in range(4):
            acc = gold(acc, yprev, ys[d], xs[d])
            yprev = ys[d]
        return dp, w, logacc, acc, yprev

    def step16(t, carry):
        for i in range(4):
            carry = quad(4 * t + i, carry)
        return carry

    carry = jax.lax.fori_loop(
        0, (S - 2) // 16, step16, (dp0, w0, m0, acc0, y0))
    for q in range(4 * ((S - 2) // 16), (S - 2) // 4):
        carry = quad(q, carry)
    dp, w, logacc, acc, yprev = carry

    for j in range(4 * ((S - 2) // 4) + 1, S - 1):
        xj = xt_ref[j]
        yj = yt_ref[pl.ds(j, 1), :]
        best, bp = _viterbi_step(dp, atb_ref, idxt_ref)
        bp_ref[j] = bp
        dp = best + xj
        s = jnp.max(w, axis=0, keepdims=True)
        w = _fwd_step(w * jnp.exp(xj), ebt_ref) * (1.0 / s)
        logacc = logacc + jnp.log(s)
        acc = gold(acc, yprev, yj, xj)
        yprev = yj

    xl = xt_ref[S - 1]
    yl = yt_ref[pl.ds(S - 1, 1), :]
    best, besti = _viterbi_step(dp, atb_ref, idxt_ref)
    bp_ref[S - 1] = besti
    dp_last = best + xl

    alpha = logacc + jnp.log(w)
    v = xl + alpha
    mz = jnp.max(v, axis=0, keepdims=True)
    z = mz + jnp.log(jnp.sum(jnp.exp(v - mz), axis=0, keepdims=True))

    acc = gold(acc, yprev, yl, xl)

    s = jnp.sum(acc, axis=0, keepdims=True)
    nll_ref[...] = jnp.sum(z - s, axis=1, keepdims=True) * (1.0 / B)

    last = jnp.zeros((1, B), jnp.int32)
    bestv = dp_last[0:1, :]
    for k in range(1, L):
        row = dp_last[k:k + 1, :]
        gt = row > bestv
        bestv = jnp.where(gt, row, bestv)
        last = jnp.where(gt, k, last)
    path_ref[pl.ds(S - 1, 1), :] = last

    def back1(t, cur):
        j = S - 1 - t
        bprow = bp_ref[j]
        prev = jnp.max(jnp.where(lane_iota == cur, bprow, 0),
                       axis=0, keepdims=True)
        path_ref[pl.ds(j - 1, 1), :] = prev
        return prev

    def back3(u, cur):
        for d in range(3):
            cur = back1(3 * u + d, cur)
        return cur

    cur = jax.lax.fori_loop(0, (S - 1) // 3, back3, last)
    for t in range(3 * ((S - 1) // 3), S - 1):
        cur = back1(t, cur)


@functools.partial(jax.jit, static_argnames=())
def kernel(x, y, A):
    xt = jnp.transpose(x, (1, 2, 0))
    yt = jnp.transpose(y, (1, 0))
    AT = jnp.transpose(A, (1, 0))

    path_t, nll = pl.pallas_call(
        _crf_kernel,
        out_shape=(
            jax.ShapeDtypeStruct((S, B), jnp.int32),
            jax.ShapeDtypeStruct((1, 1), jnp.float32),
        ),
        scratch_shapes=[
            pltpu.VMEM((S, L, B), jnp.int32),
            pltpu.VMEM((L, L, B), jnp.float32),
            pltpu.VMEM((L, L, B), jnp.float32),
            pltpu.VMEM((L, L, B), jnp.int32),
        ],
    )(xt, yt, A, AT)

    return path_t.T, nll[0, 0]
